# baseline (speedup 1.0000x reference)
import functools

import jax
import jax.numpy as jnp
from jax import lax
from jax.experimental import pallas as pl
from jax.experimental.pallas import tpu as pltpu
from jax.experimental.pallas import tpu_sc as plsc

BATCH = 16384
MF_DIM = 64
MLP_IN_HALF = 128
H0 = 128
H1 = 64

NC, NS = 2, 16
NW = NC * NS
B_PER_W = BATCH // NW
HALF = B_PER_W // 2

PACK_ROWS = 2000


def _pack_body(a_ref, b_ref, o_ref):
    o_ref[...] = jnp.concatenate([a_ref[...], b_ref[...]], axis=1)


def _pack(table):
    n = table.shape[0]
    half_blocks = (n // 2) // PACK_ROWS
    return pl.pallas_call(
        _pack_body,
        grid=(half_blocks,),
        in_specs=[
            pl.BlockSpec((PACK_ROWS, MF_DIM), lambda i: (i, 0)),
            pl.BlockSpec((PACK_ROWS, MF_DIM),
                         lambda i: (i + half_blocks, 0)),
        ],
        out_specs=pl.BlockSpec((PACK_ROWS, 2 * MF_DIM), lambda i: (i, 0)),
        out_shape=jax.ShapeDtypeStruct((n // 2, 2 * MF_DIM), jnp.float32),
    )(table, table)


def _gather_all(gmf_user_p, gmf_item_p, mlp_user, mlp_item,
                gmf_uidx, gmf_iidx, user_idxs, item_idxs):
    mesh = plsc.VectorSubcoreMesh(core_axis_name="c", subcore_axis_name="s")
    row_t = jax.ShapeDtypeStruct((BATCH, 128), jnp.float32)

    @functools.partial(
        pl.kernel,
        mesh=mesh,
        out_type=[row_t, row_t, row_t, row_t],
        scratch_types=[
            pltpu.VMEM((B_PER_W,), jnp.int32),
            pltpu.VMEM((B_PER_W,), jnp.int32),
            pltpu.VMEM((B_PER_W,), jnp.int32),
            pltpu.VMEM((B_PER_W,), jnp.int32),
            pltpu.VMEM((HALF, 128), jnp.float32),
            pltpu.VMEM((HALF, 128), jnp.float32),
            pltpu.SemaphoreType.DMA,
            pltpu.SemaphoreType.DMA,
        ],
    )
    def k(gu_hbm, gi_hbm, mu_hbm, mi_hbm,
          guidx_hbm, giidx_hbm, uidx_hbm, iidx_hbm,
          out_gu, out_gi, out_mu, out_mi,
          guidx_v, giidx_v, uidx_v, iidx_v, buf_a, buf_b, sa, sb):
        wid = lax.axis_index("s") * NC + lax.axis_index("c")
        base = wid * B_PER_W
        sl = pl.ds(base, B_PER_W)
        pltpu.sync_copy(guidx_hbm.at[sl], guidx_v)
        pltpu.sync_copy(giidx_hbm.at[sl], giidx_v)
        pltpu.sync_copy(uidx_hbm.at[sl], uidx_v)
        pltpu.sync_copy(iidx_hbm.at[sl], iidx_v)

        jobs = (
            (gu_hbm, guidx_v, out_gu),
            (gi_hbm, giidx_v, out_gi),
            (mu_hbm, uidx_v, out_mu),
            (mi_hbm, iidx_v, out_mi),
        )
        pend = []
        for tbl, idx_v, out in jobs:
            for c in (0, 1):
                buf, sem = (buf_a, sa) if c == 0 else (buf_b, sb)
                if len(pend) >= 2:
                    cpy, pbuf, pout, poff = pend.pop(0)
                    cpy.wait()
                    pltpu.sync_copy(pbuf, pout.at[pl.ds(poff, HALF)])
                cg = pltpu.async_copy(
                    tbl.at[idx_v.at[pl.ds(c * HALF, HALF)]], buf, sem)
                pend.append((cg, buf, out, base + c * HALF))
        for cpy, pbuf, pout, poff in pend:
            cpy.wait()
            pltpu.sync_copy(pbuf, pout.at[pl.ds(poff, HALF)])

    return k(gmf_user_p, gmf_item_p, mlp_user, mlp_item,
             gmf_uidx, gmf_iidx, user_idxs, item_idxs)


BT = 2048


def _head_body(gup_ref, gip_ref, mu_ref, mi_ref, pu_ref, pi_ref,
               w0u_ref, w0i_ref, b0_ref, w1_ref, b1_ref,
               wfg_ref, wfm_ref, bf_ref, out_ref):
    h0 = jnp.dot(mu_ref[...], w0u_ref[...], preferred_element_type=jnp.float32)
    h0 += jnp.dot(mi_ref[...], w0i_ref[...], preferred_element_type=jnp.float32)
    h0 = jnp.maximum(h0 + b0_ref[...], 0.0)
    h1 = jnp.dot(h0, w1_ref[...], preferred_element_type=jnp.float32)
    h1 = jnp.maximum(h1 + b1_ref[...], 0.0)
    pu = pu_ref[...]
    pi = pi_ref[...]
    gup = gup_ref[...]
    gip = gip_ref[...]
    gu = gup[:, :MF_DIM] + pu * (gup[:, MF_DIM:] - gup[:, :MF_DIM])
    gi = gip[:, :MF_DIM] + pi * (gip[:, MF_DIM:] - gip[:, :MF_DIM])
    gmf = gu * gi
    logit = jnp.sum(gmf * wfg_ref[...], axis=1) + jnp.sum(h1 * wfm_ref[...], axis=1)
    out_ref[...] = jax.nn.sigmoid(logit + bf_ref[0])


def _dense_head(gup, gip, mu, mi, pu, pi, W0, b0, W1, b1, Wf, bf):
    w0u = W0[:, :MLP_IN_HALF].T
    w0i = W0[:, MLP_IN_HALF:].T
    w1 = W1.T
    wfg = Wf[0, :MF_DIM].reshape(1, MF_DIM)
    wfm = Wf[0, MF_DIM:].reshape(1, H1)
    b0r = b0.reshape(1, H0)
    b1r = b1.reshape(1, H1)

    grid = (BATCH // BT,)
    full = lambda shape: pl.BlockSpec(shape, lambda i: (0,) * len(shape))
    return pl.pallas_call(
        _head_body,
        grid=grid,
        in_specs=[
            pl.BlockSpec((BT, 128), lambda i: (i, 0)),
            pl.BlockSpec((BT, 128), lambda i: (i, 0)),
            pl.BlockSpec((BT, MLP_IN_HALF), lambda i: (i, 0)),
            pl.BlockSpec((BT, MLP_IN_HALF), lambda i: (i, 0)),
            pl.BlockSpec((BT, 1), lambda i: (i, 0)),
            pl.BlockSpec((BT, 1), lambda i: (i, 0)),
            full((MLP_IN_HALF, H0)),
            full((MLP_IN_HALF, H0)),
            full((1, H0)),
            full((H0, H1)),
            full((1, H1)),
            full((1, MF_DIM)),
            full((1, H1)),
            full((1,)),
        ],
        out_specs=pl.BlockSpec((BT,), lambda i: (i,)),
        out_shape=jax.ShapeDtypeStruct((BATCH,), jnp.float32),
    )(gup, gip, mu, mi, pu, pi, w0u, w0i, b0r, w1, b1r, wfg, wfm, bf)


def kernel(user_idxs, item_idxs, gmf_user, gmf_item, mlp_user, mlp_item,
           W0, b0, W1, b1, Wf, bf):
    gmf_user_p = _pack(gmf_user)
    gmf_item_p = _pack(gmf_item)
    nu2 = gmf_user.shape[0] // 2
    ni2 = gmf_item.shape[0] // 2
    pu_b = user_idxs >= nu2
    pi_b = item_idxs >= ni2
    gmf_uidx = jnp.where(pu_b, user_idxs - nu2, user_idxs)
    gmf_iidx = jnp.where(pi_b, item_idxs - ni2, item_idxs)
    pu = pu_b.astype(jnp.float32).reshape(BATCH, 1)
    pi = pi_b.astype(jnp.float32).reshape(BATCH, 1)

    gup, gip, mu, mi = _gather_all(gmf_user_p, gmf_item_p, mlp_user, mlp_item,
                                   gmf_uidx, gmf_iidx, user_idxs, item_idxs)
    return _dense_head(gup, gip, mu, mi, pu, pi, W0, b0, W1, b1, Wf, bf)


# speedup vs baseline: 1.2412x; 1.2412x over previous
import functools

import jax
import jax.numpy as jnp
from jax import lax
from jax.experimental import pallas as pl
from jax.experimental.pallas import tpu as pltpu
from jax.experimental.pallas import tpu_sc as plsc

BATCH = 16384
MF_DIM = 64
MLP_IN_HALF = 128
H0 = 128
H1 = 64

NC, NS = 2, 16
NW = NC * NS
B_PER_W = BATCH // NW
HALF = B_PER_W // 2

PACK_ROWS = 2000


def _pack_body(x_ref, o_ref):
    o_ref[...] = jnp.concatenate([x_ref[0], x_ref[1]], axis=1)


def _pack(table):
    n = table.shape[0]
    half_blocks = (n // 2) // PACK_ROWS
    tv = table.reshape(2, n // 2, MF_DIM)
    return pl.pallas_call(
        _pack_body,
        grid=(half_blocks,),
        in_specs=[pl.BlockSpec((2, PACK_ROWS, MF_DIM), lambda i: (0, i, 0))],
        out_specs=pl.BlockSpec((PACK_ROWS, 2 * MF_DIM), lambda i: (i, 0)),
        out_shape=jax.ShapeDtypeStruct((n // 2, 2 * MF_DIM), jnp.float32),
    )(tv)


def _gather_all(gmf_user_p, gmf_item_p, mlp_user, mlp_item,
                gmf_uidx, gmf_iidx, user_idxs, item_idxs):
    mesh = plsc.VectorSubcoreMesh(core_axis_name="c", subcore_axis_name="s")
    row_t = jax.ShapeDtypeStruct((BATCH, 128), jnp.float32)

    @functools.partial(
        pl.kernel,
        mesh=mesh,
        out_type=[row_t, row_t, row_t, row_t],
        scratch_types=[
            pltpu.VMEM((B_PER_W,), jnp.int32),
            pltpu.VMEM((B_PER_W,), jnp.int32),
            pltpu.VMEM((B_PER_W,), jnp.int32),
            pltpu.VMEM((B_PER_W,), jnp.int32),
            pltpu.VMEM((HALF, 128), jnp.float32),
            pltpu.VMEM((HALF, 128), jnp.float32),
            pltpu.SemaphoreType.DMA,
            pltpu.SemaphoreType.DMA,
        ],
    )
    def k(gu_hbm, gi_hbm, mu_hbm, mi_hbm,
          guidx_hbm, giidx_hbm, uidx_hbm, iidx_hbm,
          out_gu, out_gi, out_mu, out_mi,
          guidx_v, giidx_v, uidx_v, iidx_v, buf_a, buf_b, sa, sb):
        wid = lax.axis_index("s") * NC + lax.axis_index("c")
        base = wid * B_PER_W
        sl = pl.ds(base, B_PER_W)
        pltpu.sync_copy(guidx_hbm.at[sl], guidx_v)
        pltpu.sync_copy(giidx_hbm.at[sl], giidx_v)
        pltpu.sync_copy(uidx_hbm.at[sl], uidx_v)
        pltpu.sync_copy(iidx_hbm.at[sl], iidx_v)

        jobs = (
            (gu_hbm, guidx_v, out_gu),
            (gi_hbm, giidx_v, out_gi),
            (mu_hbm, uidx_v, out_mu),
            (mi_hbm, iidx_v, out_mi),
        )
        pend = []
        for tbl, idx_v, out in jobs:
            for c in (0, 1):
                buf, sem = (buf_a, sa) if c == 0 else (buf_b, sb)
                if len(pend) >= 2:
                    cpy, pbuf, pout, poff = pend.pop(0)
                    cpy.wait()
                    pltpu.sync_copy(pbuf, pout.at[pl.ds(poff, HALF)])
                cg = pltpu.async_copy(
                    tbl.at[idx_v.at[pl.ds(c * HALF, HALF)]], buf, sem)
                pend.append((cg, buf, out, base + c * HALF))
        for cpy, pbuf, pout, poff in pend:
            cpy.wait()
            pltpu.sync_copy(pbuf, pout.at[pl.ds(poff, HALF)])

    return k(gmf_user_p, gmf_item_p, mlp_user, mlp_item,
             gmf_uidx, gmf_iidx, user_idxs, item_idxs)


BT = 2048


def _head_body(gup_ref, gip_ref, mu_ref, mi_ref, pu_ref, pi_ref,
               w0u_ref, w0i_ref, b0_ref, w1_ref, b1_ref,
               wfg_ref, wfm_ref, bf_ref, out_ref):
    h0 = jnp.dot(mu_ref[...], w0u_ref[...], preferred_element_type=jnp.float32)
    h0 += jnp.dot(mi_ref[...], w0i_ref[...], preferred_element_type=jnp.float32)
    h0 = jnp.maximum(h0 + b0_ref[...], 0.0)
    h1 = jnp.dot(h0, w1_ref[...], preferred_element_type=jnp.float32)
    h1 = jnp.maximum(h1 + b1_ref[...], 0.0)
    pu = pu_ref[...]
    pi = pi_ref[...]
    gup = gup_ref[...]
    gip = gip_ref[...]
    gu = gup[:, :MF_DIM] + pu * (gup[:, MF_DIM:] - gup[:, :MF_DIM])
    gi = gip[:, :MF_DIM] + pi * (gip[:, MF_DIM:] - gip[:, :MF_DIM])
    gmf = gu * gi
    logit = jnp.sum(gmf * wfg_ref[...], axis=1) + jnp.sum(h1 * wfm_ref[...], axis=1)
    out_ref[...] = jax.nn.sigmoid(logit + bf_ref[0])


def _dense_head(gup, gip, mu, mi, pu, pi, W0, b0, W1, b1, Wf, bf):
    w0u = W0[:, :MLP_IN_HALF].T
    w0i = W0[:, MLP_IN_HALF:].T
    w1 = W1.T
    wfg = Wf[0, :MF_DIM].reshape(1, MF_DIM)
    wfm = Wf[0, MF_DIM:].reshape(1, H1)
    b0r = b0.reshape(1, H0)
    b1r = b1.reshape(1, H1)

    grid = (BATCH // BT,)
    full = lambda shape: pl.BlockSpec(shape, lambda i: (0,) * len(shape))
    return pl.pallas_call(
        _head_body,
        grid=grid,
        in_specs=[
            pl.BlockSpec((BT, 128), lambda i: (i, 0)),
            pl.BlockSpec((BT, 128), lambda i: (i, 0)),
            pl.BlockSpec((BT, MLP_IN_HALF), lambda i: (i, 0)),
            pl.BlockSpec((BT, MLP_IN_HALF), lambda i: (i, 0)),
            pl.BlockSpec((BT, 1), lambda i: (i, 0)),
            pl.BlockSpec((BT, 1), lambda i: (i, 0)),
            full((MLP_IN_HALF, H0)),
            full((MLP_IN_HALF, H0)),
            full((1, H0)),
            full((H0, H1)),
            full((1, H1)),
            full((1, MF_DIM)),
            full((1, H1)),
            full((1,)),
        ],
        out_specs=pl.BlockSpec((BT,), lambda i: (i,)),
        out_shape=jax.ShapeDtypeStruct((BATCH,), jnp.float32),
    )(gup, gip, mu, mi, pu, pi, w0u, w0i, b0r, w1, b1r, wfg, wfm, bf)


def kernel(user_idxs, item_idxs, gmf_user, gmf_item, mlp_user, mlp_item,
           W0, b0, W1, b1, Wf, bf):
    gmf_user_p = _pack(gmf_user)
    gmf_item_p = _pack(gmf_item)
    nu2 = gmf_user.shape[0] // 2
    ni2 = gmf_item.shape[0] // 2
    pu_b = user_idxs >= nu2
    pi_b = item_idxs >= ni2
    gmf_uidx = jnp.where(pu_b, user_idxs - nu2, user_idxs)
    gmf_iidx = jnp.where(pi_b, item_idxs - ni2, item_idxs)
    pu = pu_b.astype(jnp.float32).reshape(BATCH, 1)
    pi = pi_b.astype(jnp.float32).reshape(BATCH, 1)

    gup, gip, mu, mi = _gather_all(gmf_user_p, gmf_item_p, mlp_user, mlp_item,
                                   gmf_uidx, gmf_iidx, user_idxs, item_idxs)
    return _dense_head(gup, gip, mu, mi, pu, pi, W0, b0, W1, b1, Wf, bf)
